# baseline (speedup 1.0000x reference)
import functools

import jax
import jax.numpy as jnp
from jax import lax
from jax.experimental import pallas as pl
from jax.experimental.pallas import tpu as pltpu
from jax.experimental.pallas import tpu_sc as plsc

EMB = 128
NBATCH = 4096
SEQ = 50
ROWS = NBATCH * SEQ

try:
    _info = plsc.get_sparse_core_info()
    _NC, _NS = int(_info.num_cores), int(_info.num_subcores)
except Exception:
    _NC, _NS = 2, 16
NW = _NC * _NS
ROWS_PER_W = ROWS // NW
CHUNK = 64
CHUNKS_PER_W = ROWS_PER_W // CHUNK
NBUF = 10
DEPTH = 6


def _make_gather():
    mesh = plsc.VectorSubcoreMesh(core_axis_name="c", subcore_axis_name="s")

    @functools.partial(
        pl.kernel,
        mesh=mesh,
        out_type=jax.ShapeDtypeStruct((ROWS, EMB), jnp.float32),
        scratch_types=[
            pltpu.VMEM((CHUNKS_PER_W, CHUNK), jnp.int32),
            [pltpu.VMEM((CHUNK, EMB), jnp.float32) for _ in range(NBUF)],
            [pltpu.SemaphoreType.DMA for _ in range(NBUF)],
            [pltpu.SemaphoreType.DMA for _ in range(NBUF)],
        ],
    )
    def gather_kernel(tok_hbm, table_hbm, out_hbm, idx_v, bufs, gsems, wsems):
        wid = lax.axis_index("s") * _NC + lax.axis_index("c")
        base = wid * ROWS_PER_W
        pltpu.sync_copy(tok_hbm.at[wid], idx_v)

        def gather(j, b):
            return pltpu.async_copy(table_hbm.at[idx_v.at[j]], bufs[b], gsems[b])

        def wback(j, b):
            return pltpu.async_copy(
                bufs[b], out_hbm.at[pl.ds(base + j * CHUNK, CHUNK)], wsems[b]
            )

        for b in range(DEPTH):
            gather(b, b)

        @pl.loop(0, CHUNKS_PER_W, step=NBUF)
        def _round(j0):
            for b in range(NBUF):
                j = j0 + b
                pltpu.make_async_copy(table_hbm.at[idx_v.at[j]], bufs[b], gsems[b]).wait()
                wback(j, b)
                jn = j + DEPTH
                bn = (b + DEPTH) % NBUF

                @pl.when(jn < CHUNKS_PER_W)
                def _():
                    @pl.when(jn >= NBUF)
                    def _():
                        pltpu.make_async_copy(
                            bufs[bn],
                            out_hbm.at[pl.ds(base + (jn - NBUF) * CHUNK, CHUNK)],
                            wsems[bn],
                        ).wait()

                    gather(jn, bn)

        for b in range(NBUF):
            j = CHUNKS_PER_W - NBUF + b
            pltpu.make_async_copy(
                bufs[b], out_hbm.at[pl.ds(base + j * CHUNK, CHUNK)], wsems[b]
            ).wait()

    return gather_kernel


_gather = _make_gather()


def kernel(tokens, emb_table):
    tok3d = tokens.T.reshape(NW, CHUNKS_PER_W, CHUNK).astype(jnp.int32)
    out = _gather(tok3d, emb_table)
    return out.reshape(SEQ, NBATCH, EMB).transpose(1, 0, 2)
